# baseline (speedup 1.0000x reference)
import functools

import jax
import jax.numpy as jnp
from jax import lax
from jax.experimental import pallas as pl
from jax.experimental.pallas import tpu as pltpu
from jax.experimental.pallas import tpu_sc as plsc

CLIP_DIM = 512
DIM_OUT = 256
BATCH = 16384

NUM_SC_CORES = 2
NUM_SC_SUBCORES = 16
NUM_WORKERS = NUM_SC_CORES * NUM_SC_SUBCORES

B_PER_W = BATCH // NUM_WORKERS
CHUNK = 128
N_CHUNKS = B_PER_W // CHUNK
NBUF = 3

L = 16
COLS = DIM_OUT // L
CBLK = 8
GROUPS_PER_CHUNK = CHUNK // L

_GATHER_DNUMS = lax.GatherDimensionNumbers(
    offset_dims=(), collapsed_slice_dims=(0,), start_index_map=(0,))


def _mlp_table_body(emb_ref, w1_ref, b1_ref, w2_ref, b2_ref, out_ref):
    h = jnp.dot(emb_ref[...], w1_ref[...], preferred_element_type=jnp.float32)
    h = jnp.maximum(h + b1_ref[...], 0.0)
    out = jnp.dot(h, w2_ref[...], preferred_element_type=jnp.float32)
    out_ref[...] = out + b2_ref[...]


def _project_table(emb, W1, b1, W2, b2):
    return pl.pallas_call(
        _mlp_table_body,
        out_shape=jax.ShapeDtypeStruct((3, DIM_OUT), jnp.float32),
    )(emb, W1, b1.reshape(1, CLIP_DIM), W2, b2.reshape(1, DIM_OUT))


def _gather_body(table_hbm, idx_hbm, out_hbm, table_v, idx_v,
                 rows0, rows1, rows2, osem0, osem1, osem2):
    wid = lax.axis_index("s") * NUM_SC_CORES + lax.axis_index("c")
    base = wid * B_PER_W

    pltpu.sync_copy(table_hbm, table_v)
    pltpu.sync_copy(idx_hbm.at[pl.ds(base, B_PER_W)], idx_v)

    rows = (rows0, rows1, rows2)
    osems = (osem0, osem1, osem2)
    writes = [None] * NBUF
    for ci in range(N_CHUNKS):
        b = ci % NBUF
        if writes[b] is not None:
            writes[b].wait()
        rows_b = rows[b]

        @plsc.parallel_loop(0, GROUPS_PER_CHUNK, unroll=2)
        def build_group(g, rows_b=rows_b, ci=ci):
            labv = idx_v[pl.ds(ci * CHUNK + g * L, L)]
            labf = [
                lax.gather(
                    labv, jnp.full((L, 1), j, jnp.int32), _GATHER_DNUMS, (1,),
                    mode=lax.GatherScatterMode.PROMISE_IN_BOUNDS
                ).astype(jnp.float32)
                for j in range(L)
            ]
            for cb in range(COLS // CBLK):
                t2 = [table_v[2, pl.ds((cb * CBLK + c) * L, L)]
                      for c in range(CBLK)]
                d0 = [table_v[0, pl.ds((cb * CBLK + c) * L, L)] - t2[c]
                      for c in range(CBLK)]
                d1 = [table_v[1, pl.ds((cb * CBLK + c) * L, L)] - t2[c]
                      for c in range(CBLK)]
                for j in range(L):
                    u0 = jnp.maximum(1.0 - labf[j], 0.0)
                    u1 = jnp.maximum(1.0 - jnp.abs(labf[j] - 1.0), 0.0)
                    row_local = g * L + j
                    for c in range(CBLK):
                        val = t2[c] + d0[c] * u0 + d1[c] * u1
                        rows_b[row_local, pl.ds((cb * CBLK + c) * L, L)] = val

        dst = out_hbm.at[pl.ds(base + ci * CHUNK, CHUNK)]
        writes[b] = pltpu.async_copy(rows_b, dst, osems[b])
    for w in writes:
        if w is not None:
            w.wait()


_sc_gather = functools.partial(
    pl.kernel,
    out_type=jax.ShapeDtypeStruct((BATCH, DIM_OUT), jnp.float32),
    mesh=plsc.VectorSubcoreMesh(
        core_axis_name="c", subcore_axis_name="s",
        num_cores=NUM_SC_CORES, num_subcores=NUM_SC_SUBCORES),
    scratch_types=[
        pltpu.VMEM((3, DIM_OUT), jnp.float32),
        pltpu.VMEM((B_PER_W,), jnp.int32),
        pltpu.VMEM((CHUNK, DIM_OUT), jnp.float32),
        pltpu.VMEM((CHUNK, DIM_OUT), jnp.float32),
        pltpu.VMEM((CHUNK, DIM_OUT), jnp.float32),
        pltpu.SemaphoreType.DMA,
        pltpu.SemaphoreType.DMA,
        pltpu.SemaphoreType.DMA,
    ],
)(_gather_body)


def kernel(class_labels, text_embeddings_raw, W1, b1, W2, b2):
    table = _project_table(text_embeddings_raw, W1, b1, W2, b2)
    labels = class_labels.astype(jnp.int32)
    return _sc_gather(table, labels)


# speedup vs baseline: 1.0932x; 1.0932x over previous
import functools

import jax
import jax.numpy as jnp
from jax import lax
from jax.experimental import pallas as pl
from jax.experimental.pallas import tpu as pltpu
from jax.experimental.pallas import tpu_sc as plsc

CLIP_DIM = 512
DIM_OUT = 256
BATCH = 16384

NUM_SC_CORES = 2
NUM_SC_SUBCORES = 16
NUM_WORKERS = NUM_SC_CORES * NUM_SC_SUBCORES

B_PER_W = BATCH // NUM_WORKERS
CHUNK = 128
N_CHUNKS = B_PER_W // CHUNK
NBUF = 3

L = 16
COLS = DIM_OUT // L
CBLK = 8
GROUPS_PER_CHUNK = CHUNK // L

_GATHER_DNUMS = lax.GatherDimensionNumbers(
    offset_dims=(), collapsed_slice_dims=(0,), start_index_map=(0,))


def _mlp_table_body(emb_ref, w1_ref, b1_ref, w2_ref, b2_ref, out_ref):
    h = jnp.dot(emb_ref[...], w1_ref[...], preferred_element_type=jnp.float32)
    h = jnp.maximum(h + b1_ref[...], 0.0)
    out = jnp.dot(h, w2_ref[...], preferred_element_type=jnp.float32)
    out_ref[...] = out + b2_ref[...]


def _project_table(emb, W1, b1, W2, b2):
    return pl.pallas_call(
        _mlp_table_body,
        out_shape=jax.ShapeDtypeStruct((3, DIM_OUT), jnp.float32),
    )(emb, W1, b1.reshape(1, CLIP_DIM), W2, b2.reshape(1, DIM_OUT))


def _gather_body(table_hbm, idx_hbm, out_hbm, table_v, idx_v,
                 rows0, rows1, rows2, osem0, osem1, osem2):
    wid = lax.axis_index("s") * NUM_SC_CORES + lax.axis_index("c")
    base = wid * B_PER_W

    pltpu.sync_copy(table_hbm, table_v)
    pltpu.sync_copy(idx_hbm.at[pl.ds(base, B_PER_W)], idx_v)

    rows = (rows0, rows1, rows2)
    osems = (osem0, osem1, osem2)
    writes = [None] * NBUF
    for ci in range(N_CHUNKS):
        b = ci % NBUF
        if writes[b] is not None:
            writes[b].wait()
        rows_b = rows[b]

        def build_group(g, _, rows_b=rows_b, ci=ci):
            labv = idx_v[pl.ds(ci * CHUNK + g * L, L)]
            labf = [
                lax.gather(
                    labv, jnp.full((L, 1), j, jnp.int32), _GATHER_DNUMS, (1,),
                    mode=lax.GatherScatterMode.PROMISE_IN_BOUNDS
                ).astype(jnp.float32)
                for j in range(L)
            ]
            for cb in range(COLS // CBLK):
                t2 = [table_v[2, pl.ds((cb * CBLK + c) * L, L)]
                      for c in range(CBLK)]
                d0 = [table_v[0, pl.ds((cb * CBLK + c) * L, L)] - t2[c]
                      for c in range(CBLK)]
                d1 = [table_v[1, pl.ds((cb * CBLK + c) * L, L)] - t2[c]
                      for c in range(CBLK)]
                for j in range(L):
                    u0 = jnp.maximum(1.0 - labf[j], 0.0)
                    u1 = jnp.maximum(1.0 - jnp.abs(labf[j] - 1.0), 0.0)
                    row_local = g * L + j
                    for c in range(CBLK):
                        val = t2[c] + d0[c] * u0 + d1[c] * u1
                        rows_b[row_local, pl.ds((cb * CBLK + c) * L, L)] = val
            return _

        lax.fori_loop(0, GROUPS_PER_CHUNK, build_group, 0, unroll=False)
        dst = out_hbm.at[pl.ds(base + ci * CHUNK, CHUNK)]
        writes[b] = pltpu.async_copy(rows_b, dst, osems[b])
    for w in writes:
        if w is not None:
            w.wait()


_sc_gather = functools.partial(
    pl.kernel,
    out_type=jax.ShapeDtypeStruct((BATCH, DIM_OUT), jnp.float32),
    mesh=plsc.VectorSubcoreMesh(
        core_axis_name="c", subcore_axis_name="s",
        num_cores=NUM_SC_CORES, num_subcores=NUM_SC_SUBCORES),
    scratch_types=[
        pltpu.VMEM((3, DIM_OUT), jnp.float32),
        pltpu.VMEM((B_PER_W,), jnp.int32),
        pltpu.VMEM((CHUNK, DIM_OUT), jnp.float32),
        pltpu.VMEM((CHUNK, DIM_OUT), jnp.float32),
        pltpu.VMEM((CHUNK, DIM_OUT), jnp.float32),
        pltpu.SemaphoreType.DMA,
        pltpu.SemaphoreType.DMA,
        pltpu.SemaphoreType.DMA,
    ],
)(_gather_body)


def kernel(class_labels, text_embeddings_raw, W1, b1, W2, b2):
    table = _project_table(text_embeddings_raw, W1, b1, W2, b2)
    labels = class_labels.astype(jnp.int32)
    return _sc_gather(table, labels)
